# baseline (speedup 1.0000x reference)
import functools

import jax
import jax.numpy as jnp
from jax import lax
from jax.experimental import pallas as pl
from jax.experimental.pallas import tpu as pltpu
from jax.experimental.pallas import tpu_sc as plsc

B, S, E = 16, 512, 256
FILT = 256
N_BINS = 256
MAXL = 2048
NC, NS = 2, 16
NW = NC * NS
FPW = (B * MAXL) // NW
CH = 128


def _a_body(sl_ref, x_ref, d_ref, pt_ref, et_ref, pemb_ref, eemb_ref,
            blo_p_ref, bhi_p_ref, blo_e_ref, bhi_e_ref,
            x2_ref, gidx_ref, mlen_ref, mask_ref):
    b = pl.program_id(0)

    @pl.when(b == B)
    def _zero_block():
        x2_ref[...] = jnp.zeros((1, S, E), jnp.float32)

    @pl.when(b < B)
    def _main():
        x = x_ref[0]
        sl = sl_ref[b]

        pt = pt_ref[0]
        oh_p = ((pt >= blo_p_ref[0][None, :]).astype(jnp.float32)
                - (pt >= bhi_p_ref[0][None, :]).astype(jnp.float32))
        et = et_ref[0]
        oh_e = ((et >= blo_e_ref[0][None, :]).astype(jnp.float32)
                - (et >= bhi_e_ref[0][None, :]).astype(jnp.float32))
        x2_ref[0] = x + oh_p @ pemb_ref[...] + oh_e @ eemb_ref[...]

        drow = d_ref[0].astype(jnp.float32)
        tokr = lax.broadcasted_iota(jnp.int32, (1, S), 1)
        dmask = jnp.where(tokr >= sl, 0.0, drow)
        ii = lax.broadcasted_iota(jnp.int32, (S, S), 0)
        jj = lax.broadcasted_iota(jnp.int32, (S, S), 1)
        cum = dmask @ (ii <= jj).astype(jnp.float32)

        total = jnp.sum(dmask).astype(jnp.int32)
        mlen = jnp.minimum(total, MAXL)
        mlen_ref[0] = jnp.full((1, 128), mlen, jnp.int32)

        frames = lax.broadcasted_iota(jnp.int32, (MAXL, 1), 0)
        gef = (cum <= frames.astype(jnp.float32)).astype(jnp.float32)
        ones = jnp.full((S, 1), 1, jnp.float32)
        idxf = lax.dot_general(gef, ones, (((1,), (0,)), ((), ())),
                               preferred_element_type=jnp.float32)
        idx = jnp.clip(idxf.astype(jnp.int32), 0, S - 1)
        zidx = B * S + (frames & (S - 1))
        gidx_ref[0] = jnp.where(frames < mlen, idx + b * S, zidx)
        mask_ref[0] = (frames >= mlen).astype(jnp.int32)


def _a_call(src_lens, x, dur3, pt3, et3, pemb, eemb,
            blo_p, bhi_p, blo_e, bhi_e):
    out_shape = (
        jax.ShapeDtypeStruct((B + 1, S, E), jnp.float32),
        jax.ShapeDtypeStruct((B, MAXL, 1), jnp.int32),
        jax.ShapeDtypeStruct((B, 1, 128), jnp.int32),
        jax.ShapeDtypeStruct((B, MAXL, 1), jnp.int32),
    )

    def full(shape):
        return pl.BlockSpec(shape, lambda b, n=len(shape): (0,) * n)

    def per_b(s1, s2):
        return pl.BlockSpec((1, s1, s2),
                            lambda b: (jnp.minimum(b, B - 1), 0, 0))

    return pl.pallas_call(
        _a_body,
        grid=(B + 1,),
        in_specs=[
            pl.BlockSpec(memory_space=pltpu.SMEM),
            per_b(S, E),
            per_b(1, S),
            per_b(S, 1),
            per_b(S, 1),
            full((N_BINS, E)), full((N_BINS, E)),
            full((1, N_BINS)), full((1, N_BINS)),
            full((1, N_BINS)), full((1, N_BINS)),
        ],
        out_specs=[
            pl.BlockSpec((1, S, E), lambda b: (b, 0, 0)),
            per_b(MAXL, 1),
            per_b(1, 128),
            per_b(MAXL, 1),
        ],
        out_shape=out_shape,
    )(src_lens, x, dur3, pt3, et3, pemb, eemb, blo_p, bhi_p, blo_e, bhi_e)


def _b_body(sl_ref, bl_ref, x_ref,
            w1_ref, b1_ref, g1_ref, be1_ref,
            w2_ref, b2_ref, g2_ref, be2_ref, wl_ref,
            logd_ref, pp_ref, ep_ref):
    b = pl.program_id(0)
    x = x_ref[0]
    sl = sl_ref[b]
    tokc = lax.broadcasted_iota(jnp.int32, (S, 1), 0)
    padc = tokc >= sl

    zrow = jnp.zeros((1, E), jnp.float32)
    xcat = jnp.concatenate(
        [jnp.concatenate([zrow, x[:-1]], axis=0), x,
         jnp.concatenate([x[1:], zrow], axis=0)], axis=1)
    xcat_b = xcat.astype(jnp.bfloat16)

    rnorm = jnp.full((FILT, 1), 1.0 / FILT, jnp.float32)

    def mmf32(a, w):
        return lax.dot_general(a, w, (((1,), (0,)), ((), ())),
                               preferred_element_type=jnp.float32)

    def ln(h, g, be):
        m = mmf32(h, rnorm)
        msq = mmf32(h * h, rnorm)
        scale = lax.rsqrt(msq - m * m + 1e-5)
        return (h - m) * scale * g[None, :] + be[None, :]

    def predictor(i, out_ref):
        h = jnp.maximum(mmf32(xcat_b, w1_ref[i]) + b1_ref[i][None, :], 0.0)
        h = ln(h, g1_ref[i], be1_ref[i]).astype(jnp.bfloat16)
        zr = jnp.zeros((1, FILT), jnp.bfloat16)
        hcat = jnp.concatenate(
            [jnp.concatenate([zr, h[:-1]], axis=0), h,
             jnp.concatenate([h[1:], zr], axis=0)], axis=1)
        h2 = jnp.maximum(mmf32(hcat, w2_ref[i]) + b2_ref[i][None, :], 0.0)
        h2 = ln(h2, g2_ref[i], be2_ref[i])
        o = jnp.sum(h2 * wl_ref[i], axis=1, keepdims=True) + bl_ref[i, 0]
        out_ref[0] = jnp.where(padc, 0.0, o)

    predictor(0, logd_ref)
    predictor(1, pp_ref)
    predictor(2, ep_ref)


def _b_call(src_lens, bls, x, W1r, b1s, g1s, be1s, W2r, b2s, g2s, be2s, Wlr):
    out_shape = (
        jax.ShapeDtypeStruct((B, S, 1), jnp.float32),
        jax.ShapeDtypeStruct((B, S, 1), jnp.float32),
        jax.ShapeDtypeStruct((B, S, 1), jnp.float32),
    )

    def full(shape):
        return pl.BlockSpec(shape, lambda b, n=len(shape): (0,) * n)

    def per_b(s1, s2):
        return pl.BlockSpec((1, s1, s2), lambda b: (b, 0, 0))

    return pl.pallas_call(
        _b_body,
        grid=(B,),
        in_specs=[
            pl.BlockSpec(memory_space=pltpu.SMEM),
            pl.BlockSpec(memory_space=pltpu.SMEM),
            per_b(S, E),
            full((3, 3 * E, FILT)),
            full((3, FILT)), full((3, FILT)), full((3, FILT)),
            full((3, 3 * FILT, FILT)),
            full((3, FILT)), full((3, FILT)), full((3, FILT)),
            full((3, 1, FILT)),
        ],
        out_specs=[per_b(S, 1), per_b(S, 1), per_b(S, 1)],
        out_shape=out_shape,
    )(src_lens, bls, x, W1r, b1s, g1s, be1s, W2r, b2s, g2s, be2s, Wlr)


def _sc_gather(x2f, gidxf):
    mesh = plsc.VectorSubcoreMesh(core_axis_name="c", subcore_axis_name="s")

    @functools.partial(
        pl.kernel,
        mesh=mesh,
        out_type=jax.ShapeDtypeStruct((B * MAXL, E), jnp.float32),
        scratch_types=[
            pltpu.VMEM((FPW,), jnp.int32),
            pltpu.VMEM((CH, E), jnp.float32),
            pltpu.VMEM((CH, E), jnp.float32),
            pltpu.SemaphoreType.DMA,
            pltpu.SemaphoreType.DMA,
        ],
    )
    def k(x2_hbm, gidx_hbm, out_hbm, idx_v, buf0, buf1, sem0, sem1):
        cid = lax.axis_index("c")
        sid = lax.axis_index("s")
        wid = sid * NC + cid
        base = wid * FPW
        pltpu.sync_copy(gidx_hbm.at[pl.ds(base, FPW)], idx_v)
        bufs = (buf0, buf1)
        sems = (sem0, sem1)
        nch = FPW // CH
        cps = [None] * nch
        cps[0] = pltpu.async_copy(x2_hbm.at[idx_v.at[pl.ds(0, CH)]],
                                  bufs[0], sems[0])
        for ci in range(nch):
            if ci + 1 < nch:
                cps[ci + 1] = pltpu.async_copy(
                    x2_hbm.at[idx_v.at[pl.ds((ci + 1) * CH, CH)]],
                    bufs[(ci + 1) % 2], sems[(ci + 1) % 2])
            cps[ci].wait()
            pltpu.sync_copy(bufs[ci % 2],
                            out_hbm.at[pl.ds(base + ci * CH, CH)])

    return k(x2f, gidxf)


def kernel(x, src_lens, duration_target, pitch_target, energy_target,
           max_len, W1s, b1s, g1s, be1s, W2s, b2s, g2s, be2s, Wls, bls,
           pitch_emb, energy_emb, pitch_bins, energy_bins):
    ninf = jnp.full((1,), -jnp.inf, jnp.float32)
    pinf = jnp.full((1,), jnp.inf, jnp.float32)
    blo_p = jnp.concatenate([ninf, pitch_bins]).reshape(1, N_BINS)
    bhi_p = jnp.concatenate([pitch_bins, pinf]).reshape(1, N_BINS)
    blo_e = jnp.concatenate([ninf, energy_bins]).reshape(1, N_BINS)
    bhi_e = jnp.concatenate([energy_bins, pinf]).reshape(1, N_BINS)

    x2p, gidx3, mlen3, mask3 = _a_call(
        src_lens, x,
        duration_target.reshape(B, 1, S),
        pitch_target.reshape(B, S, 1),
        energy_target.reshape(B, S, 1),
        pitch_emb, energy_emb, blo_p, bhi_p, blo_e, bhi_e)

    mel = _sc_gather(x2p.reshape((B + 1) * S, E), gidx3.reshape(B * MAXL))

    logd3, pp3, ep3 = _b_call(
        src_lens, bls, x,
        W1s.reshape(3, 3 * E, FILT).astype(jnp.bfloat16), b1s, g1s, be1s,
        W2s.reshape(3, 3 * FILT, FILT).astype(jnp.bfloat16), b2s, g2s, be2s,
        Wls.reshape(3, 1, FILT))

    return (mel.reshape(B, MAXL, E),
            logd3.reshape(B, S), pp3.reshape(B, S), ep3.reshape(B, S),
            mlen3[:, 0, 0], mask3.reshape(B, MAXL).astype(bool))


# speedup vs baseline: 1.6359x; 1.6359x over previous
import functools

import jax
import jax.numpy as jnp
from jax import lax
from jax.experimental import pallas as pl
from jax.experimental.pallas import tpu as pltpu
from jax.experimental.pallas import tpu_sc as plsc

B, S, E = 16, 512, 256
FILT = 256
N_BINS = 256
MAXL = 2048
NC, NS = 2, 16
NW = NC * NS
FPW = (B * MAXL) // NW
CH = 128


def _a_body(sl_ref, x_ref, d_ref, pt_ref, et_ref, pemb_ref, eemb_ref,
            blo_p_ref, bhi_p_ref, blo_e_ref, bhi_e_ref,
            x2_ref, gidx_ref, mlen_ref, mask_ref):
    b = pl.program_id(0)

    @pl.when(b == B)
    def _zero_block():
        x2_ref[...] = jnp.zeros((1, S, E), jnp.float32)

    @pl.when(b < B)
    def _main():
        r = lax.rem(b, 8)
        x = x_ref[0]
        sl = sl_ref[b]

        pt = pt_ref[pl.ds(r, 1), :]
        ohT_p = ((pt >= blo_p_ref[...]).astype(jnp.float32)
                 - (pt >= bhi_p_ref[...]).astype(jnp.float32))
        et = et_ref[pl.ds(r, 1), :]
        ohT_e = ((et >= blo_e_ref[...]).astype(jnp.float32)
                 - (et >= bhi_e_ref[...]).astype(jnp.float32))

        def dotT(ohT, emb):
            return lax.dot_general(ohT, emb, (((0,), (0,)), ((), ())),
                                   preferred_element_type=jnp.float32)

        x2_ref[0] = x + dotT(ohT_p, pemb_ref[...]) + dotT(ohT_e, eemb_ref[...])

        drow = d_ref[pl.ds(r, 1), :].astype(jnp.float32)
        tokr = lax.broadcasted_iota(jnp.int32, (1, S), 1)
        dmask = jnp.where(tokr >= sl, 0.0, drow)
        ii = lax.broadcasted_iota(jnp.int32, (S, S), 0)
        jj = lax.broadcasted_iota(jnp.int32, (S, S), 1)
        ltri = (jj <= ii).astype(jnp.float32)
        cum_col = lax.dot_general(ltri, dmask, (((1,), (1,)), ((), ())),
                                  preferred_element_type=jnp.float32)

        total = jnp.sum(dmask).astype(jnp.int32)
        mlen = jnp.minimum(total, MAXL)
        mlen_ref[0] = jnp.full((1, 128), mlen, jnp.int32)

        frames = lax.broadcasted_iota(jnp.int32, (1, MAXL), 1)
        gefT = (cum_col <= frames.astype(jnp.float32)).astype(jnp.float32)
        ones = jnp.full((1, S), 1, jnp.float32)
        idxf = lax.dot_general(ones, gefT, (((1,), (0,)), ((), ())),
                               preferred_element_type=jnp.float32)
        idx = jnp.clip(idxf.astype(jnp.int32), 0, S - 1)
        zidx = B * S + (frames & (S - 1))
        gidx_ref[pl.ds(r, 1), :] = jnp.where(frames < mlen, idx + b * S, zidx)
        mask_ref[pl.ds(r, 1), :] = (frames >= mlen).astype(jnp.int32)


def _a_call(src_lens, x, dur, pt, et, pemb, eemb,
            blo_p, bhi_p, blo_e, bhi_e):
    out_shape = (
        jax.ShapeDtypeStruct((B + 1, S, E), jnp.float32),
        jax.ShapeDtypeStruct((B, MAXL), jnp.int32),
        jax.ShapeDtypeStruct((B, 1, 128), jnp.int32),
        jax.ShapeDtypeStruct((B, MAXL), jnp.int32),
    )

    def full(shape):
        return pl.BlockSpec(shape, lambda b, n=len(shape): (0,) * n)

    def row8(n):
        return pl.BlockSpec((8, n), lambda b: (jnp.minimum(b, B - 1) // 8, 0))

    return pl.pallas_call(
        _a_body,
        grid=(B + 1,),
        in_specs=[
            pl.BlockSpec(memory_space=pltpu.SMEM),
            pl.BlockSpec((1, S, E),
                         lambda b: (jnp.minimum(b, B - 1), 0, 0)),
            row8(S),
            row8(S),
            row8(S),
            full((N_BINS, E)), full((N_BINS, E)),
            full((N_BINS, 1)), full((N_BINS, 1)),
            full((N_BINS, 1)), full((N_BINS, 1)),
        ],
        out_specs=[
            pl.BlockSpec((1, S, E), lambda b: (b, 0, 0)),
            row8(MAXL),
            pl.BlockSpec((1, 1, 128),
                         lambda b: (jnp.minimum(b, B - 1), 0, 0)),
            row8(MAXL),
        ],
        out_shape=out_shape,
    )(src_lens, x, dur, pt, et, pemb, eemb, blo_p, bhi_p, blo_e, bhi_e)


def _b_body(sl_ref, blp_ref, x_ref, w1_ref, b1_ref, beg1_ref,
            w2_ref, b2_ref, wlp_ref,
            logd_ref, pp_ref, ep_ref):
    b = pl.program_id(0)
    r = lax.rem(b, 8)
    x = x_ref[0]
    sl = sl_ref[b]
    padr = lax.broadcasted_iota(jnp.int32, (1, S), 1) >= sl

    zrow = jnp.zeros((1, E), jnp.float32)
    xcat = jnp.concatenate(
        [jnp.concatenate([zrow, x[:-1]], axis=0), x,
         jnp.concatenate([x[1:], zrow], axis=0)], axis=1)
    xcat_b = xcat.astype(jnp.bfloat16)

    def mmf32(a, w):
        return lax.dot_general(a, w, (((1,), (0,)), ((), ())),
                               preferred_element_type=jnp.float32)

    h_all = jnp.maximum(mmf32(xcat_b, w1_ref[...]) + b1_ref[...], 0.0)

    def moments(h):
        m = jnp.mean(h, axis=-1, keepdims=True)
        q = jnp.mean(h * h, axis=-1, keepdims=True)
        return m, lax.rsqrt(q - m * m + 1e-5)

    def predictor(i, out_ref):
        h = h_all[:, i * FILT:(i + 1) * FILT]
        m, s = moments(h)
        v = ((h - m) * s + beg1_ref[i][None, :]).astype(jnp.bfloat16)
        zr = jnp.zeros((1, FILT), jnp.bfloat16)
        vcat = jnp.concatenate(
            [jnp.concatenate([zr, v[:-1]], axis=0), v,
             jnp.concatenate([v[1:], zr], axis=0)], axis=1)
        h2 = jnp.maximum(mmf32(vcat, w2_ref[i]) + b2_ref[i][None, :], 0.0)
        m2, s2 = moments(h2)
        c2 = (h2 - m2) * s2
        o = lax.dot_general(wlp_ref[i], c2, (((1,), (1,)), ((), ())),
                            preferred_element_type=jnp.float32) + blp_ref[i, 0]
        out_ref[pl.ds(r, 1), :] = jnp.where(padr, 0.0, o)

    predictor(0, logd_ref)
    predictor(1, pp_ref)
    predictor(2, ep_ref)


def _b_call(src_lens, blp, x, w1all, b1all, beg1, W2p, b2s, wlp):
    out_shape = (
        jax.ShapeDtypeStruct((B, S), jnp.float32),
        jax.ShapeDtypeStruct((B, S), jnp.float32),
        jax.ShapeDtypeStruct((B, S), jnp.float32),
    )

    def full(shape):
        return pl.BlockSpec(shape, lambda b, n=len(shape): (0,) * n)

    def row8(n):
        return pl.BlockSpec((8, n), lambda b: (b // 8, 0))

    return pl.pallas_call(
        _b_body,
        grid=(B,),
        in_specs=[
            pl.BlockSpec(memory_space=pltpu.SMEM),
            pl.BlockSpec(memory_space=pltpu.SMEM),
            pl.BlockSpec((1, S, E), lambda b: (b, 0, 0)),
            full((3 * E, 3 * FILT)),
            full((1, 3 * FILT)),
            full((3, FILT)),
            full((3, 3 * FILT, FILT)),
            full((3, FILT)),
            full((3, 1, FILT)),
        ],
        out_specs=[row8(S), row8(S), row8(S)],
        out_shape=out_shape,
    )(src_lens, blp, x, w1all, b1all, beg1, W2p, b2s, wlp)


def _sc_gather(x2f, gidxf):
    mesh = plsc.VectorSubcoreMesh(core_axis_name="c", subcore_axis_name="s")

    @functools.partial(
        pl.kernel,
        mesh=mesh,
        out_type=jax.ShapeDtypeStruct((B * MAXL, E), jnp.float32),
        scratch_types=[
            pltpu.VMEM((FPW,), jnp.int32),
            pltpu.VMEM((CH, E), jnp.float32),
            pltpu.VMEM((CH, E), jnp.float32),
            pltpu.SemaphoreType.DMA,
            pltpu.SemaphoreType.DMA,
        ],
    )
    def k(x2_hbm, gidx_hbm, out_hbm, idx_v, buf0, buf1, sem0, sem1):
        cid = lax.axis_index("c")
        sid = lax.axis_index("s")
        wid = sid * NC + cid
        base = wid * FPW
        pltpu.sync_copy(gidx_hbm.at[pl.ds(base, FPW)], idx_v)
        bufs = (buf0, buf1)
        sems = (sem0, sem1)
        nch = FPW // CH
        cps = [None] * nch
        cps[0] = pltpu.async_copy(x2_hbm.at[idx_v.at[pl.ds(0, CH)]],
                                  bufs[0], sems[0])
        for ci in range(nch):
            if ci + 1 < nch:
                cps[ci + 1] = pltpu.async_copy(
                    x2_hbm.at[idx_v.at[pl.ds((ci + 1) * CH, CH)]],
                    bufs[(ci + 1) % 2], sems[(ci + 1) % 2])
            cps[ci].wait()
            pltpu.sync_copy(bufs[ci % 2],
                            out_hbm.at[pl.ds(base + ci * CH, CH)])

    return k(x2f, gidxf)


def kernel(x, src_lens, duration_target, pitch_target, energy_target,
           max_len, W1s, b1s, g1s, be1s, W2s, b2s, g2s, be2s, Wls, bls,
           pitch_emb, energy_emb, pitch_bins, energy_bins):
    ninf = jnp.full((1,), -jnp.inf, jnp.float32)
    pinf = jnp.full((1,), jnp.inf, jnp.float32)
    blo_p = jnp.concatenate([ninf, pitch_bins]).reshape(N_BINS, 1)
    bhi_p = jnp.concatenate([pitch_bins, pinf]).reshape(N_BINS, 1)
    blo_e = jnp.concatenate([ninf, energy_bins]).reshape(N_BINS, 1)
    bhi_e = jnp.concatenate([energy_bins, pinf]).reshape(N_BINS, 1)

    x2p, gidx2, mlen3, mask2 = _a_call(
        src_lens, x, duration_target, pitch_target, energy_target,
        pitch_emb, energy_emb, blo_p, bhi_p, blo_e, bhi_e)

    mel = _sc_gather(x2p.reshape((B + 1) * S, E), gidx2.reshape(B * MAXL))

    w1all = jnp.concatenate(
        [W1s[i].reshape(3 * E, FILT) for i in range(3)],
        axis=1).astype(jnp.bfloat16)
    b1all = b1s.reshape(1, 3 * FILT)
    beg1 = be1s / g1s
    W2p = (W2s * g1s[:, None, :, None]).reshape(
        3, 3 * FILT, FILT).astype(jnp.bfloat16)
    wlp = (g2s * Wls[:, :, 0]).reshape(3, 1, FILT)
    blp = bls + jnp.sum(be2s * Wls[:, :, 0], axis=1, keepdims=True)

    logd, pp, ep = _b_call(src_lens, blp, x, w1all, b1all, beg1, W2p,
                           b2s, wlp)

    return (mel.reshape(B, MAXL, E), logd, pp, ep,
            mlen3[:, 0, 0], mask2.astype(bool))
